# baseline (speedup 1.0000x reference)
import functools

import jax
import jax.numpy as jnp
from jax import lax
from jax.experimental import pallas as pl
from jax.experimental.pallas import tpu as pltpu
from jax.experimental.pallas import tpu_sc as plsc

B = 16384
D = 2049
DM = 2048
L = 16
DPAD = 2176
NC, NS = 2, 16
NW = NC * NS

BSC = 5120
BOFF = B - BSC
NG = 8
BPW = BSC // NG
NJ = BPW // L
NQ = 4
KQ = DM // NQ
KCH = 64
NCH = KQ // KCH

TB = 1024


def _sc_body(inp_t, idxa, out2, table, y_out,
             buf0, buf1, embv, out2v, yv, tlv, tmp, idxv, shared,
             sem0, sem1):
    cid = lax.axis_index("c")
    sid = lax.axis_index("s")
    g_loc = lax.rem(sid, NG // NC)
    q = lax.div(sid, NG // NC)
    b0 = BOFF + (cid * (NG // NC) + g_loc) * BPW
    kq0 = q * KQ

    pltpu.sync_copy(idxa, idxv)
    pltpu.sync_copy(out2.at[pl.ds(b0, BPW)], out2v)
    pltpu.sync_copy(inp_t.at[pl.ds(DM, 1), pl.ds(b0, BPW)], tlv)
    pltpu.async_copy(table.at[idxv], embv, sem0).wait()

    zero = jnp.zeros((L,), jnp.float32)
    z16 = jnp.zeros((L,), jnp.int32)
    emb_t = plsc.load_gather(embv, [z16, jnp.full((L,), DM, jnp.int32)])
    emb_t = emb_t * jnp.where(q == NQ - 1, 1.0, 0.0).astype(jnp.float32)

    bufs = (buf0, buf1)
    sems = (sem0, sem1)

    for b in range(2):
        pltpu.async_copy(
            inp_t.at[pl.ds(kq0 + b * KCH, KCH), pl.ds(b0, BPW)],
            bufs[b], sems[b])

    def pair(g, accs):
        for b in range(2):
            ch = 2 * g + b
            buf = bufs[b]
            sem = sems[b]
            k0 = kq0 + ch * KCH
            pltpu.make_async_copy(
                inp_t.at[pl.ds(k0, KCH), pl.ds(b0, BPW)], buf, sem).wait()

            new = []
            for half in range(2):
                sub = accs[half * (NJ // 2):(half + 1) * (NJ // 2)]

                def kstep(kk, a, _half=half):
                    ebk = plsc.load_gather(embv, [z16, z16 + (k0 + kk)])
                    off = _half * (NJ // 2) * L
                    return tuple(
                        a[j] + buf[kk, pl.ds(off + j * L, L)] * ebk
                        for j in range(NJ // 2))

                new.extend(lax.fori_loop(0, KCH, kstep, tuple(sub)))
            accs = tuple(new)

            @pl.when(ch + 2 < NCH)
            def _():
                pltpu.async_copy(
                    inp_t.at[pl.ds(k0 + 2 * KCH, KCH), pl.ds(b0, BPW)],
                    buf, sem)
        return accs

    accs = lax.fori_loop(0, NCH // 2, pair, tuple(zero for _ in range(NJ)))

    for j in range(NJ):
        yv[pl.ds(j * L, L)] = accs[j] + tlv[0, pl.ds(j * L, L)] * emb_t
    pltpu.sync_copy(yv, shared.at[sid])
    plsc.subcore_barrier()

    @pl.when(q == 0)
    def _():
        for qq in range(1, NQ):
            pltpu.sync_copy(shared.at[g_loc + qq * (NG // NC)], tmp)
            for j in range(NJ):
                yv[pl.ds(j * L, L)] = (yv[pl.ds(j * L, L)]
                                       + tmp[pl.ds(j * L, L)])
        for j in range(NJ):
            yv[pl.ds(j * L, L)] = jnp.minimum(yv[pl.ds(j * L, L)],
                                              out2v[pl.ds(j * L, L)])
        pltpu.sync_copy(
            yv, y_out.at[pl.ds((cid * (NG // NC) + g_loc) * BPW, BPW)])


_sc_call = pl.kernel(
    _sc_body,
    out_type=jax.ShapeDtypeStruct((BSC,), jnp.float32),
    mesh=plsc.VectorSubcoreMesh(core_axis_name="c", subcore_axis_name="s",
                                num_cores=NC, num_subcores=NS),
    scratch_types=[
        pltpu.VMEM((KCH, BPW), jnp.float32),
        pltpu.VMEM((KCH, BPW), jnp.float32),
        pltpu.VMEM((1, DPAD), jnp.float32),
        pltpu.VMEM((BPW,), jnp.float32),
        pltpu.VMEM((BPW,), jnp.float32),
        pltpu.VMEM((1, BPW), jnp.float32),
        pltpu.VMEM((BPW,), jnp.float32),
        pltpu.VMEM((1,), jnp.int32),
        pltpu.VMEM_SHARED((NS, BPW), jnp.float32),
        pltpu.SemaphoreType.DMA,
        pltpu.SemaphoreType.DMA,
    ],
    compiler_params=pltpu.CompilerParams(needs_layout_passes=False),
)


def _tc_body(idx_ref, x_ref, emb_ref, out2_ref, o_ref):
    ids = lax.broadcasted_iota(jnp.int32, (1, 7), 1)
    sel = (ids == idx_ref[0]).astype(jnp.float32)
    e = jax.lax.dot_general(sel, emb_ref[...], (((1,), (0,)), ((), ())),
                            preferred_element_type=jnp.float32)
    y = jax.lax.dot_general(e, x_ref[...], (((1,), (0,)), ((), ())),
                            preferred_element_type=jnp.float32)
    o_ref[...] = jnp.minimum(y, out2_ref[...])


def _tc_call(inp_t, idxa, out2_2d, emb_tab):
    return pl.pallas_call(
        _tc_body,
        out_shape=jax.ShapeDtypeStruct((1, BOFF), jnp.float32),
        grid=(BOFF // TB,),
        in_specs=[
            pl.BlockSpec(memory_space=pltpu.SMEM),
            pl.BlockSpec((D, TB), lambda i: (0, i)),
            pl.BlockSpec((7, D), lambda i: (0, 0)),
            pl.BlockSpec((1, TB), lambda i: (0, i)),
        ],
        out_specs=pl.BlockSpec((1, TB), lambda i: (0, i)),
    )(idxa, inp_t, emb_tab, out2_2d)


def _max_body(y_ref, o_ref):
    o_ref[0, 0] = jnp.max(y_ref[...])


def _final_max(y):
    return pl.pallas_call(
        _max_body,
        out_shape=jax.ShapeDtypeStruct((1, 1), jnp.float32),
        out_specs=pl.BlockSpec(memory_space=pltpu.SMEM),
    )(y.reshape(B // 128, 128))


def kernel(input, idx, out2, emb_table):
    idxa = jnp.full((1,), idx, jnp.int32)
    table = jnp.pad(emb_table, ((0, 0), (0, DPAD - D)))
    inp_t = input.T
    y_sc = _sc_call(inp_t, idxa, out2, table)
    y_tc = _tc_call(inp_t, idxa, out2.reshape(1, B), emb_table)
    y = jnp.concatenate([y_tc.reshape(BOFF), y_sc])
    out = _final_max(y).reshape(1)
    return (out, y.reshape(1, B))


# speedup vs baseline: 1.0439x; 1.0439x over previous
import functools

import jax
import jax.numpy as jnp
from jax import lax
from jax.experimental import pallas as pl
from jax.experimental.pallas import tpu as pltpu
from jax.experimental.pallas import tpu_sc as plsc

B = 16384
D = 2049
DM = 2048
L = 16
DPAD = 2176
NC, NS = 2, 16
NW = NC * NS

BSC = 5120
BOFF = B - BSC
NG = 8
BPW = BSC // NG
NJ = BPW // L
NQ = 4
KQ = DM // NQ
KCH = 64
NCH = KQ // KCH

TB = 1024


def _sc_body(inp_t, idxa, out2, table, y_out,
             buf0, buf1, embv, out2v, yv, tlv, tmp, idxv, shared,
             sem0, sem1):
    cid = lax.axis_index("c")
    sid = lax.axis_index("s")
    g_loc = lax.rem(sid, NG // NC)
    q = lax.div(sid, NG // NC)
    b0 = BOFF + (cid * (NG // NC) + g_loc) * BPW
    kq0 = q * KQ

    pltpu.sync_copy(idxa, idxv)
    pltpu.sync_copy(out2.at[pl.ds(b0, BPW)], out2v)
    pltpu.sync_copy(inp_t.at[pl.ds(DM, 1), pl.ds(b0, BPW)], tlv)
    pltpu.async_copy(table.at[idxv], embv, sem0).wait()

    zero = jnp.zeros((L,), jnp.float32)
    z16 = jnp.zeros((L,), jnp.int32)
    emb_t = plsc.load_gather(embv, [z16, jnp.full((L,), DM, jnp.int32)])
    emb_t = emb_t * jnp.where(q == NQ - 1, 1.0, 0.0).astype(jnp.float32)

    bufs = (buf0, buf1)
    sems = (sem0, sem1)

    for b in range(2):
        pltpu.async_copy(
            inp_t.at[pl.ds(kq0 + b * KCH, KCH), pl.ds(b0, BPW)],
            bufs[b], sems[b])

    def pair(g, accs):
        for b in range(2):
            ch = 2 * g + b
            buf = bufs[b]
            sem = sems[b]
            k0 = kq0 + ch * KCH
            pltpu.make_async_copy(
                inp_t.at[pl.ds(k0, KCH), pl.ds(b0, BPW)], buf, sem).wait()

            new = []
            for half in range(2):
                sub = accs[half * (NJ // 2):(half + 1) * (NJ // 2)]

                def kstep(kk, a, _half=half):
                    ebk = plsc.load_gather(embv, [z16, z16 + (k0 + kk)])
                    off = _half * (NJ // 2) * L
                    return tuple(
                        a[j] + buf[kk, pl.ds(off + j * L, L)] * ebk
                        for j in range(NJ // 2))

                new.extend(lax.fori_loop(0, KCH, kstep, tuple(sub)))
            accs = tuple(new)

            @pl.when(ch + 2 < NCH)
            def _():
                pltpu.async_copy(
                    inp_t.at[pl.ds(k0 + 2 * KCH, KCH), pl.ds(b0, BPW)],
                    buf, sem)
        return accs

    accs = lax.fori_loop(0, NCH // 2, pair, tuple(zero for _ in range(NJ)))

    for j in range(NJ):
        yv[pl.ds(j * L, L)] = accs[j] + tlv[0, pl.ds(j * L, L)] * emb_t
    pltpu.sync_copy(yv, shared.at[sid])
    plsc.subcore_barrier()

    @pl.when(q == 0)
    def _():
        for qq in range(1, NQ):
            pltpu.sync_copy(shared.at[g_loc + qq * (NG // NC)], tmp)
            for j in range(NJ):
                yv[pl.ds(j * L, L)] = (yv[pl.ds(j * L, L)]
                                       + tmp[pl.ds(j * L, L)])
        for j in range(NJ):
            yv[pl.ds(j * L, L)] = jnp.minimum(yv[pl.ds(j * L, L)],
                                              out2v[pl.ds(j * L, L)])
        pltpu.sync_copy(
            yv, y_out.at[pl.ds((cid * (NG // NC) + g_loc) * BPW, BPW)])


_sc_call = pl.kernel(
    _sc_body,
    out_type=jax.ShapeDtypeStruct((BSC,), jnp.float32),
    mesh=plsc.VectorSubcoreMesh(core_axis_name="c", subcore_axis_name="s",
                                num_cores=NC, num_subcores=NS),
    scratch_types=[
        pltpu.VMEM((KCH, BPW), jnp.float32),
        pltpu.VMEM((KCH, BPW), jnp.float32),
        pltpu.VMEM((1, DPAD), jnp.float32),
        pltpu.VMEM((BPW,), jnp.float32),
        pltpu.VMEM((BPW,), jnp.float32),
        pltpu.VMEM((1, BPW), jnp.float32),
        pltpu.VMEM((BPW,), jnp.float32),
        pltpu.VMEM((1,), jnp.int32),
        pltpu.VMEM_SHARED((NS, BPW), jnp.float32),
        pltpu.SemaphoreType.DMA,
        pltpu.SemaphoreType.DMA,
    ],
    compiler_params=pltpu.CompilerParams(needs_layout_passes=False,
                                         skip_device_barrier=True,
                                         disable_semaphore_checks=True,
                                         disable_bounds_checks=True),
)


def _tc_body(idx_ref, x_ref, emb_ref, out2_ref, o_ref):
    ids = lax.broadcasted_iota(jnp.int32, (1, 7), 1)
    sel = (ids == idx_ref[0]).astype(jnp.float32)
    e = jax.lax.dot_general(sel, emb_ref[...], (((1,), (0,)), ((), ())),
                            preferred_element_type=jnp.float32)
    y = jax.lax.dot_general(e, x_ref[...], (((1,), (0,)), ((), ())),
                            preferred_element_type=jnp.float32)
    o_ref[...] = jnp.minimum(y, out2_ref[...])


def _tc_call(inp_t, idxa, out2_2d, emb_tab):
    return pl.pallas_call(
        _tc_body,
        out_shape=jax.ShapeDtypeStruct((1, BOFF), jnp.float32),
        grid=(BOFF // TB,),
        in_specs=[
            pl.BlockSpec(memory_space=pltpu.SMEM),
            pl.BlockSpec((D, TB), lambda i: (0, i)),
            pl.BlockSpec((7, D), lambda i: (0, 0)),
            pl.BlockSpec((1, TB), lambda i: (0, i)),
        ],
        out_specs=pl.BlockSpec((1, TB), lambda i: (0, i)),
    )(idxa, inp_t, emb_tab, out2_2d)


def _max_body(y_ref, o_ref):
    o_ref[0, 0] = jnp.max(y_ref[...])


def _final_max(y):
    return pl.pallas_call(
        _max_body,
        out_shape=jax.ShapeDtypeStruct((1, 1), jnp.float32),
        out_specs=pl.BlockSpec(memory_space=pltpu.SMEM),
    )(y.reshape(B // 128, 128))


def kernel(input, idx, out2, emb_table):
    idxa = jnp.full((1,), idx, jnp.int32)
    table = jnp.pad(emb_table, ((0, 0), (0, DPAD - D)))
    inp_t = input.T
    y_sc = _sc_call(inp_t, idxa, out2, table)
    y_tc = _tc_call(inp_t, idxa, out2.reshape(1, B), emb_table)
    y = jnp.concatenate([y_tc.reshape(BOFF), y_sc])
    out = _final_max(y).reshape(1)
    return (out, y.reshape(1, B))
